# baseline (speedup 1.0000x reference)
import functools

import jax
import jax.numpy as jnp
from jax import lax
from jax.experimental import pallas as pl
from jax.experimental.pallas import tpu as pltpu
from jax.experimental.pallas import tpu_sc as plsc

S, B, D, H, E, K, DFF = 2048, 2, 1024, 16, 8, 2, 2048
T = S * B
DH = D // H
C = (S * B * K * 5) // (E * 4)
NW = 32
TPW = T // NW
CH = 64
NCH = TPW // CH
TRASH = E * C
BUF_ROWS = E * C + 256


def _qkv_body(x_ref, w_ref, b_ref, o_ref):
    bsel = pl.program_id(2)
    x = jnp.where(bsel == 0, x_ref[:, 0, :], x_ref[:, 1, :])
    acc = lax.dot_general(x, w_ref[...], (((1,), (1,)), ((), ())),
                          preferred_element_type=jnp.float32)
    o_ref[0, 0, 0] = acc + b_ref[0, 0][None, :]


def _qkv_proj(src3d, in_proj_w, in_proj_b):
    BS = 512
    grid = (S // BS, 3, B, H)
    return pl.pallas_call(
        _qkv_body,
        grid=grid,
        in_specs=[
            pl.BlockSpec((BS, B, D), lambda i, q, b, h: (i, 0, 0)),
            pl.BlockSpec((DH, D), lambda i, q, b, h: (q * H + h, 0)),
            pl.BlockSpec((1, 1, DH), lambda i, q, b, h: (q * H + h, 0, 0)),
        ],
        out_specs=pl.BlockSpec((1, 1, 1, BS, DH),
                               lambda i, q, b, h: (q, b, h, i, 0)),
        out_shape=jax.ShapeDtypeStruct((3, B, H, S, DH), jnp.float32),
    )(src3d, in_proj_w, in_proj_b.reshape(3 * H, 1, DH))


def _attn_body(q_ref, k_ref, v_ref, o_ref):
    q = q_ref[0, 0, 0]
    k = k_ref[0, 0, 0]
    v = v_ref[0, 0, 0]
    s = lax.dot_general(q, k, (((1,), (1,)), ((), ())),
                        preferred_element_type=jnp.float32)
    s = s * (1.0 / (DH ** 0.5))
    m = jnp.max(s, axis=-1, keepdims=True)
    e = jnp.exp(s - m)
    a = e / jnp.sum(e, axis=-1, keepdims=True)
    o_ref[0, 0, 0] = lax.dot_general(a, v, (((1,), (0,)), ((), ())),
                                     preferred_element_type=jnp.float32)


def _attention(qkv):
    BS = 256
    grid = (B, H, S // BS)
    return pl.pallas_call(
        _attn_body,
        grid=grid,
        in_specs=[
            pl.BlockSpec((1, 1, 1, BS, DH), lambda b, h, i: (0, b, h, i, 0)),
            pl.BlockSpec((1, 1, 1, S, DH), lambda b, h, i: (1, b, h, 0, 0)),
            pl.BlockSpec((1, 1, 1, S, DH), lambda b, h, i: (2, b, h, 0, 0)),
        ],
        out_specs=pl.BlockSpec((1, 1, 1, BS, DH),
                               lambda b, h, i: (0, b, h, i, 0)),
        out_shape=jax.ShapeDtypeStruct((1, B, H, S, DH), jnp.float32),
    )(qkv, qkv, qkv)


def _proj_ln1_body(a_ref, src_ref, w_ref, b_ref, g_ref, bb_ref, wg_ref,
                   x1_ref, lg_ref, acc_ref):
    h = pl.program_id(2)
    bsel = pl.program_id(0)
    part = lax.dot_general(a_ref[0, 0, 0], w_ref[0], (((1,), (0,)), ((), ())),
                           preferred_element_type=jnp.float32)

    @pl.when(h == 0)
    def _():
        acc_ref[...] = part

    @pl.when(h > 0)
    def _():
        acc_ref[...] = acc_ref[...] + part

    @pl.when(h == H - 1)
    def _():
        srcb = jnp.where(bsel == 0, src_ref[:, 0, :], src_ref[:, 1, :])
        y = acc_ref[...] + b_ref[...] + srcb
        mu = jnp.mean(y, axis=-1, keepdims=True)
        d = y - mu
        var = jnp.mean(d * d, axis=-1, keepdims=True)
        x1 = d * lax.rsqrt(var + 1e-5) * g_ref[...] + bb_ref[...]
        x1_ref[...] = x1
        lg_ref[...] = lax.dot_general(x1, wg_ref[...],
                                      (((1,), (0,)), ((), ())),
                                      preferred_element_type=jnp.float32)


def _proj_ln1(attn, src3d, woT, out_b, ln1_g, ln1_b, Wg):
    BR = 256
    nsb = S // BR
    grid = (B, nsb, H)
    return pl.pallas_call(
        _proj_ln1_body,
        grid=grid,
        in_specs=[
            pl.BlockSpec((1, 1, 1, BR, DH), lambda b, i, h: (0, b, h, i, 0)),
            pl.BlockSpec((BR, B, D), lambda b, i, h: (i, 0, 0)),
            pl.BlockSpec((1, DH, D), lambda b, i, h: (h, 0, 0)),
            pl.BlockSpec((1, D), lambda b, i, h: (0, 0)),
            pl.BlockSpec((1, D), lambda b, i, h: (0, 0)),
            pl.BlockSpec((1, D), lambda b, i, h: (0, 0)),
            pl.BlockSpec((D, E), lambda b, i, h: (0, 0)),
        ],
        out_specs=[
            pl.BlockSpec((BR, D), lambda b, i, h: (b * nsb + i, 0)),
            pl.BlockSpec((BR, E), lambda b, i, h: (b * nsb + i, 0)),
        ],
        out_shape=[
            jax.ShapeDtypeStruct((T, D), jnp.float32),
            jax.ShapeDtypeStruct((T, E), jnp.float32),
        ],
        scratch_shapes=[pltpu.VMEM((BR, D), jnp.float32)],
    )(attn, src3d, woT, out_b.reshape(1, D), ln1_g.reshape(1, D),
      ln1_b.reshape(1, D), Wg)


def _top2(lg, lanes):
    m = jnp.max(lg, axis=-1, keepdims=True)
    ex = jnp.exp(lg - m)
    p = ex / jnp.sum(ex, axis=-1, keepdims=True)
    m1 = jnp.max(p, axis=-1, keepdims=True)
    i1 = jnp.min(jnp.where(p == m1, lanes, E), axis=-1, keepdims=True)
    p2 = jnp.where(lanes == i1, -1.0, p)
    m2 = jnp.max(p2, axis=-1, keepdims=True)
    i2 = jnp.min(jnp.where(p2 == m2, lanes, E), axis=-1, keepdims=True)
    gs = m1 + m2
    oh1 = (lanes == i1).astype(jnp.float32)
    oh2 = (lanes == i2).astype(jnp.float32)
    return m1 / gs, m2 / gs, oh1, oh2


def _router_body(lg_ref, g0_ref, g1_ref, dst0_ref, dst1_ref, src0_ref,
                 src1_ref, kp0_ref, kp1_ref, acc_ref):
    i = pl.program_id(0)

    @pl.when(i == 0)
    def _():
        acc_ref[...] = jnp.zeros_like(acc_ref)

    BR = lg_ref.shape[1]
    lanes = lax.broadcasted_iota(jnp.int32, (BR, E), 1)
    ga0, gb0, oha0, ohb0 = _top2(lg_ref[0], lanes)
    ga1, gb1, oha1, ohb1 = _top2(lg_ref[1], lanes)

    gsum = oha0 + ohb0 + oha1 + ohb1
    r = lax.broadcasted_iota(jnp.int32, (BR, BR), 0)
    c = lax.broadcasted_iota(jnp.int32, (BR, BR), 1)
    lstrict = (c < r).astype(jnp.float32)
    f = lax.dot_general(lstrict, gsum, (((1,), (0,)), ((), ())),
                        preferred_element_type=jnp.float32) + acc_ref[...]
    corr = oha0 + ohb0

    def emit(b, oh1, oh2, g0, g1, fb):
        pos0 = jnp.sum(fb * oh1, axis=-1, keepdims=True).astype(jnp.int32)
        pos1 = jnp.sum(fb * oh2, axis=-1, keepdims=True).astype(jnp.int32)
        i1 = jnp.sum(
            lanes * oh1.astype(jnp.int32), axis=-1, keepdims=True)
        i2 = jnp.sum(
            lanes * oh2.astype(jnp.int32), axis=-1, keepdims=True)
        kp0 = (pos0 < C).astype(jnp.int32)
        kp1 = (pos1 < C).astype(jnp.int32)
        s0 = i1 * C + jnp.minimum(pos0, C - 1)
        s1 = i2 * C + jnp.minimum(pos1, C - 1)
        g0_ref[b] = g0
        g1_ref[b] = g1
        kp0_ref[b] = kp0
        kp1_ref[b] = kp1
        src0_ref[b] = s0
        src1_ref[b] = s1
        dst0_ref[b] = jnp.where(kp0 > 0, s0, TRASH)
        dst1_ref[b] = jnp.where(kp1 > 0, s1, TRASH)

    emit(0, oha0, ohb0, ga0, gb0, f)
    emit(1, oha1, ohb1, ga1, gb1, f + corr)

    acc_ref[...] = acc_ref[...] + jnp.sum(gsum, axis=0, keepdims=True)


def _router(logits2):
    BR = 128
    grid = (S // BR,)
    spec_d = pl.BlockSpec((B, BR, 1), lambda i: (0, i, 0))
    f32 = jax.ShapeDtypeStruct((B, S, 1), jnp.float32)
    i32 = jax.ShapeDtypeStruct((B, S, 1), jnp.int32)
    return pl.pallas_call(
        _router_body,
        grid=grid,
        in_specs=[pl.BlockSpec((B, BR, E), lambda i: (0, i, 0))],
        out_specs=[spec_d] * 8,
        out_shape=[f32, f32, i32, i32, i32, i32, i32, i32],
        scratch_shapes=[pltpu.VMEM((1, E), jnp.float32)],
    )(logits2)


def _sc_dispatch(x1, dst_idx):
    mesh = plsc.VectorSubcoreMesh(core_axis_name="c", subcore_axis_name="s")

    @functools.partial(
        pl.kernel,
        out_type=jax.ShapeDtypeStruct((BUF_ROWS, D), jnp.float32),
        mesh=mesh,
        scratch_types=[
            pltpu.VMEM((2 * NCH, CH), jnp.int32),
            pltpu.VMEM((CH, D), jnp.float32),
            pltpu.SemaphoreType.DMA,
        ],
    )
    def k(x1_hbm, dst_hbm, buf_hbm, idx_v, rows_v, sem):
        nc = 2
        wid = lax.axis_index("s") * nc + lax.axis_index("c")
        base = wid * TPW
        pltpu.sync_copy(dst_hbm.at[wid], idx_v)
        for c in range(NCH):
            pltpu.sync_copy(x1_hbm.at[pl.ds(base + c * CH, CH)], rows_v)
            pltpu.async_copy(rows_v, buf_hbm.at[idx_v.at[c]], sem).wait()
            pltpu.async_copy(rows_v, buf_hbm.at[idx_v.at[NCH + c]], sem).wait()

    return k(x1, dst_idx)


def _sc_gather(ob, src_idx):
    mesh = plsc.VectorSubcoreMesh(core_axis_name="c", subcore_axis_name="s")

    @functools.partial(
        pl.kernel,
        out_type=jax.ShapeDtypeStruct((2 * T, D), jnp.float32),
        mesh=mesh,
        scratch_types=[
            pltpu.VMEM((2 * NCH, CH), jnp.int32),
            pltpu.VMEM((CH, D), jnp.float32),
            pltpu.SemaphoreType.DMA,
        ],
    )
    def k(ob_hbm, src_hbm, comb_hbm, idx_v, rows_v, sem):
        nc = 2
        wid = lax.axis_index("s") * nc + lax.axis_index("c")
        base = wid * TPW
        pltpu.sync_copy(src_hbm.at[wid], idx_v)
        for kk in range(2):
            for c in range(NCH):
                pltpu.async_copy(ob_hbm.at[idx_v.at[kk * NCH + c]],
                                 rows_v, sem).wait()
                pltpu.sync_copy(
                    rows_v, comb_hbm.at[pl.ds(kk * T + base + c * CH, CH)])

    return k(ob, src_idx)


def _ffn_body(x_ref, w1_ref, b1_ref, w2_ref, b2_ref, o_ref):
    fb = pl.program_id(2)
    h = lax.dot_general(x_ref[...], w1_ref[0], (((1,), (0,)), ((), ())),
                        preferred_element_type=jnp.float32)
    h = jnp.maximum(h + b1_ref[0], 0.0)
    part = lax.dot_general(h, w2_ref[0], (((1,), (0,)), ((), ())),
                           preferred_element_type=jnp.float32)

    @pl.when(fb == 0)
    def _():
        o_ref[...] = part + b2_ref[0]

    @pl.when(fb > 0)
    def _():
        o_ref[...] = o_ref[...] + part


def _ffn(buf, W1, b1, W2, b2):
    BC, BF = 256, 512
    ncb, nfb = C // BC, DFF // BF
    grid = (E, ncb, nfb)
    return pl.pallas_call(
        _ffn_body,
        grid=grid,
        in_specs=[
            pl.BlockSpec((BC, D), lambda e, cb, fb: (e * ncb + cb, 0)),
            pl.BlockSpec((1, D, BF), lambda e, cb, fb: (e, 0, fb)),
            pl.BlockSpec((1, 1, BF), lambda e, cb, fb: (e, 0, fb)),
            pl.BlockSpec((1, BF, D), lambda e, cb, fb: (e, fb, 0)),
            pl.BlockSpec((1, 1, D), lambda e, cb, fb: (e, 0, 0)),
        ],
        out_specs=pl.BlockSpec((BC, D), lambda e, cb, fb: (e * ncb + cb, 0)),
        out_shape=jax.ShapeDtypeStruct((E * C, D), jnp.float32),
    )(buf, W1, b1.reshape(E, 1, DFF), W2, b2.reshape(E, 1, D))


def _combine_body(x1a_ref, x1b_ref, c0a_ref, c0b_ref, c1a_ref, c1b_ref,
                  g0_ref, g1_ref, k0_ref, k1_ref, g_ref, b_ref, o_ref):
    def side(b, x1_ref, c0_ref, c1_ref):
        x1 = x1_ref[...]
        m0 = jnp.where(k0_ref[b] > 0, g0_ref[b] * c0_ref[...], 0.0)
        m1 = jnp.where(k1_ref[b] > 0, g1_ref[b] * c1_ref[...], 0.0)
        y = x1 + m0 + m1
        mu = jnp.mean(y, axis=-1, keepdims=True)
        d = y - mu
        var = jnp.mean(d * d, axis=-1, keepdims=True)
        return d * lax.rsqrt(var + 1e-5) * g_ref[...] + b_ref[...]

    y0 = side(0, x1a_ref, c0a_ref, c1a_ref)
    y1 = side(1, x1b_ref, c0b_ref, c1b_ref)
    o_ref[...] = jnp.concatenate([y0[:, None, :], y1[:, None, :]], axis=1)


def _combine_ln2(x1, comb, g0, g1, kp0, kp1, ln2_g, ln2_b):
    BR = 128
    nb = S // BR
    grid = (nb,)
    spec_d = pl.BlockSpec((B, BR, 1), lambda i: (0, i, 0))
    spec_row = lambda blk: pl.BlockSpec((BR, D), lambda i, b=blk: (b + i, 0))
    return pl.pallas_call(
        _combine_body,
        grid=grid,
        in_specs=[
            spec_row(0), spec_row(nb),
            spec_row(0), spec_row(nb),
            spec_row(2 * nb), spec_row(3 * nb),
            spec_d, spec_d, spec_d, spec_d,
            pl.BlockSpec((1, D), lambda i: (0, 0)),
            pl.BlockSpec((1, D), lambda i: (0, 0)),
        ],
        out_specs=pl.BlockSpec((BR, B, D), lambda i: (i, 0, 0)),
        out_shape=jax.ShapeDtypeStruct((S, B, D), jnp.float32),
    )(x1, x1, comb, comb, comb, comb, g0, g1, kp0, kp1,
      ln2_g.reshape(1, D), ln2_b.reshape(1, D))


def kernel(src, in_proj_w, in_proj_b, out_w, out_b, ln1_g, ln1_b, ln2_g,
           ln2_b, Wg, W1, b1, W2, b2):
    qkv = _qkv_proj(src, in_proj_w, in_proj_b)
    attn = _attention(qkv)
    woT = out_w.T.reshape(H, DH, D)
    x1, logits = _proj_ln1(attn, src, woT, out_b, ln1_g, ln1_b, Wg)
    g0, g1, dst0, dst1, src0, src1, kp0, kp1 = _router(
        logits.reshape(B, S, E))

    def chunked(a):
        return a.reshape(NW, NCH, CH)

    dst_idx = jnp.concatenate([chunked(dst0), chunked(dst1)], axis=1)
    src_idx = jnp.concatenate([chunked(src0), chunked(src1)], axis=1)

    buf = _sc_dispatch(x1, dst_idx)
    ob = _ffn(buf, W1, b1, W2, b2)
    comb = _sc_gather(ob, src_idx)
    return _combine_ln2(x1, comb, g0, g1, kp0, kp1, ln2_g, ln2_b)


# speedup vs baseline: 1.3359x; 1.3359x over previous
import functools

import jax
import jax.numpy as jnp
from jax import lax
from jax.experimental import pallas as pl
from jax.experimental.pallas import tpu as pltpu
from jax.experimental.pallas import tpu_sc as plsc

S, B, D, H, E, K, DFF = 2048, 2, 1024, 16, 8, 2, 2048
T = S * B
DH = D // H
C = (S * B * K * 5) // (E * 4)
NW = 32
TPW = T // NW
CH = 64
NCH = TPW // CH
TRASH = E * C
BUF_ROWS = E * C + 256


def _qkv_body(x_ref, w_ref, b_ref, o_ref):
    bsel = pl.program_id(2)
    x = jnp.where(bsel == 0, x_ref[:, 0, :], x_ref[:, 1, :])
    acc = lax.dot_general(x, w_ref[...], (((1,), (1,)), ((), ())),
                          preferred_element_type=jnp.float32)
    o_ref[0, 0, 0] = acc + b_ref[0, 0][None, :]


def _qkv_proj(src3d, in_proj_w, in_proj_b):
    BS = 512
    grid = (S // BS, 3, B, H)
    return pl.pallas_call(
        _qkv_body,
        grid=grid,
        in_specs=[
            pl.BlockSpec((BS, B, D), lambda i, q, b, h: (i, 0, 0)),
            pl.BlockSpec((DH, D), lambda i, q, b, h: (q * H + h, 0)),
            pl.BlockSpec((1, 1, DH), lambda i, q, b, h: (q * H + h, 0, 0)),
        ],
        out_specs=pl.BlockSpec((1, 1, 1, BS, DH),
                               lambda i, q, b, h: (q, b, h, i, 0)),
        out_shape=jax.ShapeDtypeStruct((3, B, H, S, DH), jnp.float32),
    )(src3d, in_proj_w, in_proj_b.reshape(3 * H, 1, DH))


def _attn_body(q_ref, k_ref, v_ref, o_ref):
    q = q_ref[0, 0, 0]
    k = k_ref[0, 0, 0]
    v = v_ref[0, 0, 0]
    s = lax.dot_general(q, k, (((1,), (1,)), ((), ())),
                        preferred_element_type=jnp.float32)
    s = s * (1.0 / (DH ** 0.5))
    m = jnp.max(s, axis=-1, keepdims=True)
    e = jnp.exp(s - m)
    a = e / jnp.sum(e, axis=-1, keepdims=True)
    o_ref[0, 0, 0] = lax.dot_general(a, v, (((1,), (0,)), ((), ())),
                                     preferred_element_type=jnp.float32)


def _attention(qkv):
    BS = 256
    grid = (B, H, S // BS)
    return pl.pallas_call(
        _attn_body,
        grid=grid,
        in_specs=[
            pl.BlockSpec((1, 1, 1, BS, DH), lambda b, h, i: (0, b, h, i, 0)),
            pl.BlockSpec((1, 1, 1, S, DH), lambda b, h, i: (1, b, h, 0, 0)),
            pl.BlockSpec((1, 1, 1, S, DH), lambda b, h, i: (2, b, h, 0, 0)),
        ],
        out_specs=pl.BlockSpec((1, 1, 1, BS, DH),
                               lambda b, h, i: (0, b, h, i, 0)),
        out_shape=jax.ShapeDtypeStruct((1, B, H, S, DH), jnp.float32),
    )(qkv, qkv, qkv)


def _proj_ln1_body(a_ref, src_ref, w_ref, b_ref, g_ref, bb_ref, wg_ref,
                   x1_ref, lg_ref):
    bsel = pl.program_id(0)
    a2 = jnp.concatenate([a_ref[0, 0, h] for h in range(H)], axis=1)
    y = lax.dot_general(a2, w_ref[...], (((1,), (0,)), ((), ())),
                        preferred_element_type=jnp.float32)
    srcb = jnp.where(bsel == 0, src_ref[:, 0, :], src_ref[:, 1, :])
    y = y + b_ref[...] + srcb
    mu = jnp.mean(y, axis=-1, keepdims=True)
    d = y - mu
    var = jnp.mean(d * d, axis=-1, keepdims=True)
    x1 = d * lax.rsqrt(var + 1e-5) * g_ref[...] + bb_ref[...]
    x1_ref[...] = x1
    lg_ref[...] = lax.dot_general(x1, wg_ref[...],
                                  (((1,), (0,)), ((), ())),
                                  preferred_element_type=jnp.float32)


def _proj_ln1(attn, src3d, woT2, out_b, ln1_g, ln1_b, Wg):
    BR = 256
    nsb = S // BR
    grid = (B, nsb)
    return pl.pallas_call(
        _proj_ln1_body,
        grid=grid,
        in_specs=[
            pl.BlockSpec((1, 1, H, BR, DH), lambda b, i: (0, b, 0, i, 0)),
            pl.BlockSpec((BR, B, D), lambda b, i: (i, 0, 0)),
            pl.BlockSpec((D, D), lambda b, i: (0, 0)),
            pl.BlockSpec((1, D), lambda b, i: (0, 0)),
            pl.BlockSpec((1, D), lambda b, i: (0, 0)),
            pl.BlockSpec((1, D), lambda b, i: (0, 0)),
            pl.BlockSpec((D, E), lambda b, i: (0, 0)),
        ],
        out_specs=[
            pl.BlockSpec((BR, D), lambda b, i: (b * nsb + i, 0)),
            pl.BlockSpec((BR, E), lambda b, i: (b * nsb + i, 0)),
        ],
        out_shape=[
            jax.ShapeDtypeStruct((T, D), jnp.float32),
            jax.ShapeDtypeStruct((T, E), jnp.float32),
        ],
    )(attn, src3d, woT2, out_b.reshape(1, D), ln1_g.reshape(1, D),
      ln1_b.reshape(1, D), Wg)


def _top2(lg, lanes):
    m = jnp.max(lg, axis=-1, keepdims=True)
    ex = jnp.exp(lg - m)
    p = ex / jnp.sum(ex, axis=-1, keepdims=True)
    m1 = jnp.max(p, axis=-1, keepdims=True)
    i1 = jnp.min(jnp.where(p == m1, lanes, E), axis=-1, keepdims=True)
    p2 = jnp.where(lanes == i1, -1.0, p)
    m2 = jnp.max(p2, axis=-1, keepdims=True)
    i2 = jnp.min(jnp.where(p2 == m2, lanes, E), axis=-1, keepdims=True)
    gs = m1 + m2
    oh1 = (lanes == i1).astype(jnp.float32)
    oh2 = (lanes == i2).astype(jnp.float32)
    return m1 / gs, m2 / gs, oh1, oh2


def _router_body(lg_ref, g0_ref, g1_ref, dst0_ref, dst1_ref, src0_ref,
                 src1_ref, kp0_ref, kp1_ref, acc_ref):
    i = pl.program_id(0)

    @pl.when(i == 0)
    def _():
        acc_ref[...] = jnp.zeros_like(acc_ref)

    BR = lg_ref.shape[1]
    lanes = lax.broadcasted_iota(jnp.int32, (BR, E), 1)
    ga0, gb0, oha0, ohb0 = _top2(lg_ref[0], lanes)
    ga1, gb1, oha1, ohb1 = _top2(lg_ref[1], lanes)

    gsum = oha0 + ohb0 + oha1 + ohb1
    r = lax.broadcasted_iota(jnp.int32, (BR, BR), 0)
    c = lax.broadcasted_iota(jnp.int32, (BR, BR), 1)
    lstrict = (c < r).astype(jnp.float32)
    f = lax.dot_general(lstrict, gsum, (((1,), (0,)), ((), ())),
                        preferred_element_type=jnp.float32) + acc_ref[...]
    corr = oha0 + ohb0

    def emit(b, oh1, oh2, g0, g1, fb):
        pos0 = jnp.sum(fb * oh1, axis=-1, keepdims=True).astype(jnp.int32)
        pos1 = jnp.sum(fb * oh2, axis=-1, keepdims=True).astype(jnp.int32)
        i1 = jnp.sum(
            lanes * oh1.astype(jnp.int32), axis=-1, keepdims=True)
        i2 = jnp.sum(
            lanes * oh2.astype(jnp.int32), axis=-1, keepdims=True)
        kp0 = (pos0 < C).astype(jnp.int32)
        kp1 = (pos1 < C).astype(jnp.int32)
        s0 = i1 * C + jnp.minimum(pos0, C - 1)
        s1 = i2 * C + jnp.minimum(pos1, C - 1)
        g0_ref[b] = g0
        g1_ref[b] = g1
        kp0_ref[b] = kp0
        kp1_ref[b] = kp1
        src0_ref[b] = s0
        src1_ref[b] = s1
        dst0_ref[b] = jnp.where(kp0 > 0, s0, TRASH)
        dst1_ref[b] = jnp.where(kp1 > 0, s1, TRASH)

    emit(0, oha0, ohb0, ga0, gb0, f)
    emit(1, oha1, ohb1, ga1, gb1, f + corr)

    acc_ref[...] = acc_ref[...] + jnp.sum(gsum, axis=0, keepdims=True)


def _router(logits2):
    BR = 512
    grid = (S // BR,)
    spec_d = pl.BlockSpec((B, BR, 1), lambda i: (0, i, 0))
    f32 = jax.ShapeDtypeStruct((B, S, 1), jnp.float32)
    i32 = jax.ShapeDtypeStruct((B, S, 1), jnp.int32)
    return pl.pallas_call(
        _router_body,
        grid=grid,
        in_specs=[pl.BlockSpec((B, BR, E), lambda i: (0, i, 0))],
        out_specs=[spec_d] * 8,
        out_shape=[f32, f32, i32, i32, i32, i32, i32, i32],
        scratch_shapes=[pltpu.VMEM((1, E), jnp.float32)],
    )(logits2)


def _sc_dispatch(x1, dst_idx):
    mesh = plsc.VectorSubcoreMesh(core_axis_name="c", subcore_axis_name="s")

    @functools.partial(
        pl.kernel,
        out_type=jax.ShapeDtypeStruct((BUF_ROWS, D), jnp.float32),
        mesh=mesh,
        scratch_types=[
            pltpu.VMEM((2 * NCH, CH), jnp.int32),
            pltpu.VMEM((CH, D), jnp.float32),
            pltpu.SemaphoreType.DMA,
        ],
    )
    def k(x1_hbm, dst_hbm, buf_hbm, idx_v, rows_v, sem):
        nc = 2
        wid = lax.axis_index("s") * nc + lax.axis_index("c")
        base = wid * TPW
        pltpu.sync_copy(dst_hbm.at[wid], idx_v)
        for c in range(NCH):
            pltpu.sync_copy(x1_hbm.at[pl.ds(base + c * CH, CH)], rows_v)
            pltpu.async_copy(rows_v, buf_hbm.at[idx_v.at[c]], sem).wait()
            pltpu.async_copy(rows_v, buf_hbm.at[idx_v.at[NCH + c]], sem).wait()

    return k(x1, dst_idx)


def _sc_gather(ob, src_idx):
    mesh = plsc.VectorSubcoreMesh(core_axis_name="c", subcore_axis_name="s")

    @functools.partial(
        pl.kernel,
        out_type=jax.ShapeDtypeStruct((2 * T, D), jnp.float32),
        mesh=mesh,
        scratch_types=[
            pltpu.VMEM((2 * NCH, CH), jnp.int32),
            pltpu.VMEM((CH, D), jnp.float32),
            pltpu.SemaphoreType.DMA,
        ],
    )
    def k(ob_hbm, src_hbm, comb_hbm, idx_v, rows_v, sem):
        nc = 2
        wid = lax.axis_index("s") * nc + lax.axis_index("c")
        base = wid * TPW
        pltpu.sync_copy(src_hbm.at[wid], idx_v)
        for kk in range(2):
            for c in range(NCH):
                pltpu.async_copy(ob_hbm.at[idx_v.at[kk * NCH + c]],
                                 rows_v, sem).wait()
                pltpu.sync_copy(
                    rows_v, comb_hbm.at[pl.ds(kk * T + base + c * CH, CH)])

    return k(ob, src_idx)


def _ffn_body(x_ref, w1_ref, b1_ref, w2_ref, b2_ref, o_ref):
    fb = pl.program_id(1)
    h = lax.dot_general(x_ref[...], w1_ref[0], (((1,), (0,)), ((), ())),
                        preferred_element_type=jnp.float32)
    h = jnp.maximum(h + b1_ref[0], 0.0)
    part = lax.dot_general(h, w2_ref[0], (((1,), (0,)), ((), ())),
                           preferred_element_type=jnp.float32)

    @pl.when(fb == 0)
    def _():
        o_ref[...] = part + b2_ref[0]

    @pl.when(fb > 0)
    def _():
        o_ref[...] = o_ref[...] + part


def _ffn(buf, W1, b1, W2, b2):
    BF = 1024
    nfb = DFF // BF
    grid = (E, nfb)
    return pl.pallas_call(
        _ffn_body,
        grid=grid,
        in_specs=[
            pl.BlockSpec((C, D), lambda e, fb: (e, 0)),
            pl.BlockSpec((1, D, BF), lambda e, fb: (e, 0, fb)),
            pl.BlockSpec((1, 1, BF), lambda e, fb: (e, 0, fb)),
            pl.BlockSpec((1, BF, D), lambda e, fb: (e, fb, 0)),
            pl.BlockSpec((1, 1, D), lambda e, fb: (e, 0, 0)),
        ],
        out_specs=pl.BlockSpec((C, D), lambda e, fb: (e, 0)),
        out_shape=jax.ShapeDtypeStruct((E * C, D), jnp.float32),
    )(buf, W1, b1.reshape(E, 1, DFF), W2, b2.reshape(E, 1, D))


def _combine_body(x1a_ref, x1b_ref, c0a_ref, c0b_ref, c1a_ref, c1b_ref,
                  g0_ref, g1_ref, k0_ref, k1_ref, g_ref, b_ref, o_ref):
    def side(b, x1_ref, c0_ref, c1_ref):
        x1 = x1_ref[...]
        m0 = jnp.where(k0_ref[b] > 0, g0_ref[b] * c0_ref[...], 0.0)
        m1 = jnp.where(k1_ref[b] > 0, g1_ref[b] * c1_ref[...], 0.0)
        y = x1 + m0 + m1
        mu = jnp.mean(y, axis=-1, keepdims=True)
        d = y - mu
        var = jnp.mean(d * d, axis=-1, keepdims=True)
        return d * lax.rsqrt(var + 1e-5) * g_ref[...] + b_ref[...]

    y0 = side(0, x1a_ref, c0a_ref, c1a_ref)
    y1 = side(1, x1b_ref, c0b_ref, c1b_ref)
    o_ref[...] = jnp.concatenate([y0[:, None, :], y1[:, None, :]], axis=1)


def _combine_ln2(x1, comb, g0, g1, kp0, kp1, ln2_g, ln2_b):
    BR = 128
    nb = S // BR
    grid = (nb,)
    spec_d = pl.BlockSpec((B, BR, 1), lambda i: (0, i, 0))
    spec_row = lambda blk: pl.BlockSpec((BR, D), lambda i, b=blk: (b + i, 0))
    return pl.pallas_call(
        _combine_body,
        grid=grid,
        in_specs=[
            spec_row(0), spec_row(nb),
            spec_row(0), spec_row(nb),
            spec_row(2 * nb), spec_row(3 * nb),
            spec_d, spec_d, spec_d, spec_d,
            pl.BlockSpec((1, D), lambda i: (0, 0)),
            pl.BlockSpec((1, D), lambda i: (0, 0)),
        ],
        out_specs=pl.BlockSpec((BR, B, D), lambda i: (i, 0, 0)),
        out_shape=jax.ShapeDtypeStruct((S, B, D), jnp.float32),
    )(x1, x1, comb, comb, comb, comb, g0, g1, kp0, kp1,
      ln2_g.reshape(1, D), ln2_b.reshape(1, D))


def kernel(src, in_proj_w, in_proj_b, out_w, out_b, ln1_g, ln1_b, ln2_g,
           ln2_b, Wg, W1, b1, W2, b2):
    qkv = _qkv_proj(src, in_proj_w, in_proj_b)
    attn = _attention(qkv)
    x1, logits = _proj_ln1(attn, src, out_w.T, out_b, ln1_g, ln1_b, Wg)
    g0, g1, dst0, dst1, src0, src1, kp0, kp1 = _router(
        logits.reshape(B, S, E))

    def chunked(a):
        return a.reshape(NW, NCH, CH)

    dst_idx = jnp.concatenate([chunked(dst0), chunked(dst1)], axis=1)
    src_idx = jnp.concatenate([chunked(src0), chunked(src1)], axis=1)

    buf = _sc_dispatch(x1, dst_idx)
    ob = _ffn(buf, W1, b1, W2, b2)
    comb = _sc_gather(ob, src_idx)
    return _combine_ln2(x1, comb, g0, g1, kp0, kp1, ln2_g, ln2_b)


# speedup vs baseline: 2.2193x; 1.6613x over previous
import functools

import jax
import jax.numpy as jnp
from jax import lax
from jax.experimental import pallas as pl
from jax.experimental.pallas import tpu as pltpu
from jax.experimental.pallas import tpu_sc as plsc

S, B, D, H, E, K, DFF = 2048, 2, 1024, 16, 8, 2, 2048
T = S * B
DH = D // H
C = (S * B * K * 5) // (E * 4)
NW = 32
TPW = T // NW
CH = 32
NCH = TPW // CH
TRASH = E * C
BUF_ROWS = E * C + 256


def _qkv_body(x_ref, w_ref, b_ref, o_ref):
    bsel = pl.program_id(1)
    x = jnp.where(bsel == 0, x_ref[:, 0, :], x_ref[:, 1, :])
    acc = lax.dot_general(x, w_ref[...], (((1,), (1,)), ((), ())),
                          preferred_element_type=jnp.float32)
    o_ref[...] = acc + b_ref[...]


def _qkv_proj(src3d, in_proj_w, in_proj_b):
    BS, BN = 512, 1024
    nsb = S // BS
    grid = (3 * D // BN, B, nsb)
    return pl.pallas_call(
        _qkv_body,
        grid=grid,
        in_specs=[
            pl.BlockSpec((BS, B, D), lambda j, b, i: (i, 0, 0)),
            pl.BlockSpec((BN, D), lambda j, b, i: (j, 0)),
            pl.BlockSpec((1, BN), lambda j, b, i: (0, j)),
        ],
        out_specs=pl.BlockSpec((BS, BN), lambda j, b, i: (b * nsb + i, j)),
        out_shape=jax.ShapeDtypeStruct((T, 3 * D), jnp.float32),
    )(src3d, in_proj_w, in_proj_b.reshape(1, 3 * D))


def _attn_body(blk_ref, o_ref):
    i = pl.program_id(0)
    BS = o_ref.shape[0]
    outs = []
    for h in range(H):
        q = blk_ref[pl.ds(i * BS, BS), h * DH:(h + 1) * DH]
        k = blk_ref[:, D + h * DH:D + (h + 1) * DH]
        v = blk_ref[:, 2 * D + h * DH:2 * D + (h + 1) * DH]
        s = lax.dot_general(q, k, (((1,), (1,)), ((), ())),
                            preferred_element_type=jnp.float32)
        s = s * (1.0 / (DH ** 0.5))
        m = jnp.max(s, axis=-1, keepdims=True)
        e = jnp.exp(s - m)
        a = e / jnp.sum(e, axis=-1, keepdims=True)
        outs.append(lax.dot_general(a, v, (((1,), (0,)), ((), ())),
                                    preferred_element_type=jnp.float32))
    o_ref[...] = jnp.concatenate(outs, axis=1)


def _attention(qkv_u):
    BS = 256
    nsb = S // BS
    outs = []
    for b in range(B):
        blk = lax.slice_in_dim(qkv_u, b * S, (b + 1) * S, axis=0)
        outs.append(pl.pallas_call(
            _attn_body,
            grid=(nsb,),
            in_specs=[pl.BlockSpec((S, 3 * D), lambda i: (0, 0))],
            out_specs=pl.BlockSpec((BS, D), lambda i: (i, 0)),
            out_shape=jax.ShapeDtypeStruct((S, D), jnp.float32),
        )(blk))
    return jnp.concatenate(outs, axis=0)


def _proj_ln1_body(a_ref, src_ref, w_ref, b_ref, g_ref, bb_ref, wg_ref,
                   x1_ref, lg_ref):
    bsel = pl.program_id(0)
    y = lax.dot_general(a_ref[...], w_ref[...], (((1,), (1,)), ((), ())),
                        preferred_element_type=jnp.float32)
    srcb = jnp.where(bsel == 0, src_ref[:, 0, :], src_ref[:, 1, :])
    y = y + b_ref[...] + srcb
    mu = jnp.mean(y, axis=-1, keepdims=True)
    d = y - mu
    var = jnp.mean(d * d, axis=-1, keepdims=True)
    x1 = d * lax.rsqrt(var + 1e-5) * g_ref[...] + bb_ref[...]
    x1_ref[...] = x1
    lg_ref[...] = lax.dot_general(x1, wg_ref[...],
                                  (((1,), (0,)), ((), ())),
                                  preferred_element_type=jnp.float32)


def _proj_ln1(attn, src3d, out_w, out_b, ln1_g, ln1_b, Wg):
    BR = 256
    nsb = S // BR
    grid = (B, nsb)
    return pl.pallas_call(
        _proj_ln1_body,
        grid=grid,
        in_specs=[
            pl.BlockSpec((BR, D), lambda b, i: (b * nsb + i, 0)),
            pl.BlockSpec((BR, B, D), lambda b, i: (i, 0, 0)),
            pl.BlockSpec((D, D), lambda b, i: (0, 0)),
            pl.BlockSpec((1, D), lambda b, i: (0, 0)),
            pl.BlockSpec((1, D), lambda b, i: (0, 0)),
            pl.BlockSpec((1, D), lambda b, i: (0, 0)),
            pl.BlockSpec((D, E), lambda b, i: (0, 0)),
        ],
        out_specs=[
            pl.BlockSpec((BR, D), lambda b, i: (b * nsb + i, 0)),
            pl.BlockSpec((BR, E), lambda b, i: (b * nsb + i, 0)),
        ],
        out_shape=[
            jax.ShapeDtypeStruct((T, D), jnp.float32),
            jax.ShapeDtypeStruct((T, E), jnp.float32),
        ],
    )(attn, src3d, out_w, out_b.reshape(1, D), ln1_g.reshape(1, D),
      ln1_b.reshape(1, D), Wg)


def _top2(lg, lanes):
    m = jnp.max(lg, axis=-1, keepdims=True)
    ex = jnp.exp(lg - m)
    p = ex / jnp.sum(ex, axis=-1, keepdims=True)
    m1 = jnp.max(p, axis=-1, keepdims=True)
    i1 = jnp.min(jnp.where(p == m1, lanes, E), axis=-1, keepdims=True)
    p2 = jnp.where(lanes == i1, -1.0, p)
    m2 = jnp.max(p2, axis=-1, keepdims=True)
    i2 = jnp.min(jnp.where(p2 == m2, lanes, E), axis=-1, keepdims=True)
    gs = m1 + m2
    oh1 = (lanes == i1).astype(jnp.float32)
    oh2 = (lanes == i2).astype(jnp.float32)
    return m1 / gs, m2 / gs, oh1, oh2


def _router_body(lg_ref, g0_ref, g1_ref, dst0_ref, dst1_ref, src0_ref,
                 src1_ref, kp0_ref, kp1_ref, acc_ref):
    i = pl.program_id(0)

    @pl.when(i == 0)
    def _():
        acc_ref[...] = jnp.zeros_like(acc_ref)

    BR = lg_ref.shape[1]
    lanes = lax.broadcasted_iota(jnp.int32, (BR, E), 1)
    ga0, gb0, oha0, ohb0 = _top2(lg_ref[0], lanes)
    ga1, gb1, oha1, ohb1 = _top2(lg_ref[1], lanes)

    gsum = oha0 + ohb0 + oha1 + ohb1
    r = lax.broadcasted_iota(jnp.int32, (BR, BR), 0)
    c = lax.broadcasted_iota(jnp.int32, (BR, BR), 1)
    lstrict = (c < r).astype(jnp.float32)
    f = lax.dot_general(lstrict, gsum, (((1,), (0,)), ((), ())),
                        preferred_element_type=jnp.float32) + acc_ref[...]
    corr = oha0 + ohb0

    def emit(b, oh1, oh2, g0, g1, fb):
        pos0 = jnp.sum(fb * oh1, axis=-1, keepdims=True).astype(jnp.int32)
        pos1 = jnp.sum(fb * oh2, axis=-1, keepdims=True).astype(jnp.int32)
        i1 = jnp.sum(
            lanes * oh1.astype(jnp.int32), axis=-1, keepdims=True)
        i2 = jnp.sum(
            lanes * oh2.astype(jnp.int32), axis=-1, keepdims=True)
        kp0 = (pos0 < C).astype(jnp.int32)
        kp1 = (pos1 < C).astype(jnp.int32)
        s0 = i1 * C + jnp.minimum(pos0, C - 1)
        s1 = i2 * C + jnp.minimum(pos1, C - 1)
        g0_ref[b] = g0
        g1_ref[b] = g1
        kp0_ref[b] = kp0
        kp1_ref[b] = kp1
        src0_ref[b] = s0
        src1_ref[b] = s1
        dst0_ref[b] = jnp.where(kp0 > 0, s0, TRASH)
        dst1_ref[b] = jnp.where(kp1 > 0, s1, TRASH)

    emit(0, oha0, ohb0, ga0, gb0, f)
    emit(1, oha1, ohb1, ga1, gb1, f + corr)

    acc_ref[...] = acc_ref[...] + jnp.sum(gsum, axis=0, keepdims=True)


def _router(logits2):
    BR = 512
    grid = (S // BR,)
    spec_d = pl.BlockSpec((B, BR, 1), lambda i: (0, i, 0))
    f32 = jax.ShapeDtypeStruct((B, S, 1), jnp.float32)
    i32 = jax.ShapeDtypeStruct((B, S, 1), jnp.int32)
    return pl.pallas_call(
        _router_body,
        grid=grid,
        in_specs=[pl.BlockSpec((B, BR, E), lambda i: (0, i, 0))],
        out_specs=[spec_d] * 8,
        out_shape=[f32, f32, i32, i32, i32, i32, i32, i32],
        scratch_shapes=[pltpu.VMEM((1, E), jnp.float32)],
    )(logits2)


def _sc_dispatch(x1, dst_idx):
    mesh = plsc.VectorSubcoreMesh(core_axis_name="c", subcore_axis_name="s")

    @functools.partial(
        pl.kernel,
        out_type=jax.ShapeDtypeStruct((BUF_ROWS, D), jnp.float32),
        mesh=mesh,
        scratch_types=[
            pltpu.VMEM((2 * NCH, CH), jnp.int32),
            pltpu.VMEM((3, CH, D), jnp.float32),
            pltpu.SemaphoreType.DMA,
            pltpu.SemaphoreType.DMA,
        ],
    )
    def k(x1_hbm, dst_hbm, buf_hbm, idx_v, rows_v, lsem, ssem):
        nc = 2
        wid = lax.axis_index("s") * nc + lax.axis_index("c")
        base = wid * TPW
        pltpu.sync_copy(dst_hbm.at[wid], idx_v)
        loads = {}
        scats = {}
        for c in range(min(3, NCH)):
            loads[c] = pltpu.async_copy(
                x1_hbm.at[pl.ds(base + c * CH, CH)], rows_v.at[c % 3], lsem)
        for c in range(NCH):
            b = c % 3
            if c >= 3:
                for h in scats[c - 3]:
                    h.wait()
                loads[c] = pltpu.async_copy(
                    x1_hbm.at[pl.ds(base + c * CH, CH)], rows_v.at[b], lsem)
            loads[c].wait()
            scats[c] = (
                pltpu.async_copy(rows_v.at[b], buf_hbm.at[idx_v.at[c]], ssem),
                pltpu.async_copy(rows_v.at[b], buf_hbm.at[idx_v.at[NCH + c]],
                                 ssem),
            )
        for c in range(max(0, NCH - 3), NCH):
            for h in scats[c]:
                h.wait()

    return k(x1, dst_idx)


def _sc_gather(ob, src_idx):
    mesh = plsc.VectorSubcoreMesh(core_axis_name="c", subcore_axis_name="s")

    @functools.partial(
        pl.kernel,
        out_type=jax.ShapeDtypeStruct((2 * T, D), jnp.float32),
        mesh=mesh,
        scratch_types=[
            pltpu.VMEM((2 * NCH, CH), jnp.int32),
            pltpu.VMEM((3, CH, D), jnp.float32),
            pltpu.SemaphoreType.DMA,
            pltpu.SemaphoreType.DMA,
        ],
    )
    def k(ob_hbm, src_hbm, comb_hbm, idx_v, rows_v, gsem, wsem):
        nc = 2
        wid = lax.axis_index("s") * nc + lax.axis_index("c")
        base = wid * TPW
        pltpu.sync_copy(src_hbm.at[wid], idx_v)
        ntot = 2 * NCH
        gaths = {}
        writes = {}

        def off(c):
            return (c // NCH) * T + base + (c % NCH) * CH

        for c in range(min(3, ntot)):
            gaths[c] = pltpu.async_copy(ob_hbm.at[idx_v.at[c]],
                                        rows_v.at[c % 3], gsem)
        for c in range(ntot):
            b = c % 3
            if c >= 3:
                writes[c - 3].wait()
                gaths[c] = pltpu.async_copy(ob_hbm.at[idx_v.at[c]],
                                            rows_v.at[b], gsem)
            gaths[c].wait()
            writes[c] = pltpu.async_copy(
                rows_v.at[b], comb_hbm.at[pl.ds(off(c), CH)], wsem)
        for c in range(max(0, ntot - 3), ntot):
            writes[c].wait()

    return k(ob, src_idx)


def _ffn_body(x_ref, w1_ref, b1_ref, w2_ref, b2_ref, o_ref):
    fb = pl.program_id(1)
    h = lax.dot_general(x_ref[...], w1_ref[0], (((1,), (0,)), ((), ())),
                        preferred_element_type=jnp.float32)
    h = jnp.maximum(h + b1_ref[0], 0.0)
    part = lax.dot_general(h, w2_ref[0], (((1,), (0,)), ((), ())),
                           preferred_element_type=jnp.float32)

    @pl.when(fb == 0)
    def _():
        o_ref[...] = part + b2_ref[0]

    @pl.when(fb > 0)
    def _():
        o_ref[...] = o_ref[...] + part


def _ffn(buf, W1, b1, W2, b2):
    BF = 1024
    nfb = DFF // BF
    grid = (E, nfb)
    return pl.pallas_call(
        _ffn_body,
        grid=grid,
        in_specs=[
            pl.BlockSpec((C, D), lambda e, fb: (e, 0)),
            pl.BlockSpec((1, D, BF), lambda e, fb: (e, 0, fb)),
            pl.BlockSpec((1, 1, BF), lambda e, fb: (e, 0, fb)),
            pl.BlockSpec((1, BF, D), lambda e, fb: (e, fb, 0)),
            pl.BlockSpec((1, 1, D), lambda e, fb: (e, 0, 0)),
        ],
        out_specs=pl.BlockSpec((C, D), lambda e, fb: (e, 0)),
        out_shape=jax.ShapeDtypeStruct((E * C, D), jnp.float32),
    )(buf, W1, b1.reshape(E, 1, DFF), W2, b2.reshape(E, 1, D))


def _combine_body(x1a_ref, x1b_ref, c0a_ref, c0b_ref, c1a_ref, c1b_ref,
                  g0_ref, g1_ref, k0_ref, k1_ref, g_ref, b_ref, o_ref):
    def side(b, x1_ref, c0_ref, c1_ref):
        x1 = x1_ref[...]
        m0 = jnp.where(k0_ref[b] > 0, g0_ref[b] * c0_ref[...], 0.0)
        m1 = jnp.where(k1_ref[b] > 0, g1_ref[b] * c1_ref[...], 0.0)
        y = x1 + m0 + m1
        mu = jnp.mean(y, axis=-1, keepdims=True)
        d = y - mu
        var = jnp.mean(d * d, axis=-1, keepdims=True)
        return d * lax.rsqrt(var + 1e-5) * g_ref[...] + b_ref[...]

    y0 = side(0, x1a_ref, c0a_ref, c1a_ref)
    y1 = side(1, x1b_ref, c0b_ref, c1b_ref)
    o_ref[...] = jnp.concatenate([y0[:, None, :], y1[:, None, :]], axis=1)


def _combine_ln2(x1, comb, g0, g1, kp0, kp1, ln2_g, ln2_b):
    BR = 128
    nb = S // BR
    grid = (nb,)
    spec_d = pl.BlockSpec((B, BR, 1), lambda i: (0, i, 0))
    spec_row = lambda blk: pl.BlockSpec((BR, D), lambda i, b=blk: (b + i, 0))
    return pl.pallas_call(
        _combine_body,
        grid=grid,
        in_specs=[
            spec_row(0), spec_row(nb),
            spec_row(0), spec_row(nb),
            spec_row(2 * nb), spec_row(3 * nb),
            spec_d, spec_d, spec_d, spec_d,
            pl.BlockSpec((1, D), lambda i: (0, 0)),
            pl.BlockSpec((1, D), lambda i: (0, 0)),
        ],
        out_specs=pl.BlockSpec((BR, B, D), lambda i: (i, 0, 0)),
        out_shape=jax.ShapeDtypeStruct((S, B, D), jnp.float32),
    )(x1, x1, comb, comb, comb, comb, g0, g1, kp0, kp1,
      ln2_g.reshape(1, D), ln2_b.reshape(1, D))


def kernel(src, in_proj_w, in_proj_b, out_w, out_b, ln1_g, ln1_b, ln2_g,
           ln2_b, Wg, W1, b1, W2, b2):
    qkv = _qkv_proj(src, in_proj_w, in_proj_b)
    attn = _attention(qkv)
    x1, logits = _proj_ln1(attn, src, out_w, out_b, ln1_g, ln1_b, Wg)
    g0, g1, dst0, dst1, src0, src1, kp0, kp1 = _router(
        logits.reshape(B, S, E))

    def chunked(a):
        return a.reshape(NW, NCH, CH)

    dst_idx = jnp.concatenate([chunked(dst0), chunked(dst1)], axis=1)
    src_idx = jnp.concatenate([chunked(src0), chunked(src1)], axis=1)

    buf = _sc_dispatch(x1, dst_idx)
    ob = _ffn(buf, W1, b1, W2, b2)
    comb = _sc_gather(ob, src_idx)
    return _combine_ln2(x1, comb, g0, g1, kp0, kp1, ln2_g, ln2_b)


# speedup vs baseline: 2.7718x; 1.2489x over previous
import functools

import jax
import jax.numpy as jnp
from jax import lax
from jax.experimental import pallas as pl
from jax.experimental.pallas import tpu as pltpu
from jax.experimental.pallas import tpu_sc as plsc

S, B, D, H, E, K, DFF = 2048, 2, 1024, 16, 8, 2, 2048
T = S * B
DH = D // H
C = (S * B * K * 5) // (E * 4)
NW = 32
TPW = T // NW
CH = 32
NCH = TPW // CH
TRASH = E * C
BUF_ROWS = E * C + 256


def _qkv_body(x_ref, w_ref, b_ref, o_ref):
    bsel = pl.program_id(1)
    x = jnp.where(bsel == 0, x_ref[:, 0, :], x_ref[:, 1, :])
    acc = lax.dot_general(x, w_ref[...], (((1,), (1,)), ((), ())),
                          preferred_element_type=jnp.float32)
    o_ref[...] = acc + b_ref[...]


def _qkv_proj(src3d, in_proj_w, in_proj_b):
    BS, BN = 512, 1024
    nsb = S // BS
    grid = (3 * D // BN, B, nsb)
    return pl.pallas_call(
        _qkv_body,
        grid=grid,
        in_specs=[
            pl.BlockSpec((BS, B, D), lambda j, b, i: (i, 0, 0)),
            pl.BlockSpec((BN, D), lambda j, b, i: (j, 0)),
            pl.BlockSpec((1, BN), lambda j, b, i: (0, j)),
        ],
        out_specs=pl.BlockSpec((BS, BN), lambda j, b, i: (b * nsb + i, j)),
        out_shape=jax.ShapeDtypeStruct((T, 3 * D), jnp.float32),
    )(src3d, in_proj_w, in_proj_b.reshape(1, 3 * D))


def _attn_body(blk_ref, o_ref):
    i = pl.program_id(1)
    BS = o_ref.shape[0]
    outs = []
    for h in range(H):
        q = blk_ref[pl.ds(i * BS, BS), h * DH:(h + 1) * DH]
        k = blk_ref[:, D + h * DH:D + (h + 1) * DH]
        v = blk_ref[:, 2 * D + h * DH:2 * D + (h + 1) * DH]
        s = lax.dot_general(q * (1.0 / (DH ** 0.5)), k,
                            (((1,), (1,)), ((), ())),
                            preferred_element_type=jnp.float32)
        e = jnp.exp(s)
        a = e * (1.0 / jnp.sum(e, axis=-1, keepdims=True))
        outs.append(lax.dot_general(a, v, (((1,), (0,)), ((), ())),
                                    preferred_element_type=jnp.float32))
    o_ref[...] = jnp.concatenate(outs, axis=1)


def _attention(qkv_u):
    BS = 256
    nsb = S // BS
    return pl.pallas_call(
        _attn_body,
        grid=(B, nsb),
        in_specs=[pl.BlockSpec((S, 3 * D), lambda b, i: (b, 0))],
        out_specs=pl.BlockSpec((BS, D), lambda b, i: (b * nsb + i, 0)),
        out_shape=jax.ShapeDtypeStruct((T, D), jnp.float32),
    )(qkv_u)


def _proj_ln1_body(a_ref, src_ref, w_ref, b_ref, g_ref, bb_ref, wg_ref,
                   x1_ref, lg_ref):
    bsel = pl.program_id(0)
    y = lax.dot_general(a_ref[...], w_ref[...], (((1,), (1,)), ((), ())),
                        preferred_element_type=jnp.float32)
    srcb = jnp.where(bsel == 0, src_ref[:, 0, :], src_ref[:, 1, :])
    y = y + b_ref[...] + srcb
    mu = jnp.mean(y, axis=-1, keepdims=True)
    d = y - mu
    var = jnp.mean(d * d, axis=-1, keepdims=True)
    x1 = d * lax.rsqrt(var + 1e-5) * g_ref[...] + bb_ref[...]
    x1_ref[...] = x1
    lg_ref[...] = lax.dot_general(x1, wg_ref[...],
                                  (((1,), (0,)), ((), ())),
                                  preferred_element_type=jnp.float32)


def _proj_ln1(attn, src3d, out_w, out_b, ln1_g, ln1_b, Wg):
    BR = 256
    nsb = S // BR
    grid = (B, nsb)
    return pl.pallas_call(
        _proj_ln1_body,
        grid=grid,
        in_specs=[
            pl.BlockSpec((BR, D), lambda b, i: (b * nsb + i, 0)),
            pl.BlockSpec((BR, B, D), lambda b, i: (i, 0, 0)),
            pl.BlockSpec((D, D), lambda b, i: (0, 0)),
            pl.BlockSpec((1, D), lambda b, i: (0, 0)),
            pl.BlockSpec((1, D), lambda b, i: (0, 0)),
            pl.BlockSpec((1, D), lambda b, i: (0, 0)),
            pl.BlockSpec((D, E), lambda b, i: (0, 0)),
        ],
        out_specs=[
            pl.BlockSpec((BR, D), lambda b, i: (b * nsb + i, 0)),
            pl.BlockSpec((BR, E), lambda b, i: (b * nsb + i, 0)),
        ],
        out_shape=[
            jax.ShapeDtypeStruct((T, D), jnp.float32),
            jax.ShapeDtypeStruct((T, E), jnp.float32),
        ],
    )(attn, src3d, out_w, out_b.reshape(1, D), ln1_g.reshape(1, D),
      ln1_b.reshape(1, D), Wg)


def _top2(lg, lanes):
    m = jnp.max(lg, axis=-1, keepdims=True)
    ex = jnp.exp(lg - m)
    p = ex / jnp.sum(ex, axis=-1, keepdims=True)
    m1 = jnp.max(p, axis=-1, keepdims=True)
    i1 = jnp.min(jnp.where(p == m1, lanes, E), axis=-1, keepdims=True)
    p2 = jnp.where(lanes == i1, -1.0, p)
    m2 = jnp.max(p2, axis=-1, keepdims=True)
    i2 = jnp.min(jnp.where(p2 == m2, lanes, E), axis=-1, keepdims=True)
    gs = m1 + m2
    oh1 = (lanes == i1).astype(jnp.float32)
    oh2 = (lanes == i2).astype(jnp.float32)
    return m1 / gs, m2 / gs, oh1, oh2


def _router_body(lg_ref, g0_ref, g1_ref, dst0_ref, dst1_ref, src0_ref,
                 src1_ref, kp0_ref, kp1_ref, acc_ref):
    i = pl.program_id(0)

    @pl.when(i == 0)
    def _():
        acc_ref[...] = jnp.zeros_like(acc_ref)

    BR = lg_ref.shape[1]
    lanes = lax.broadcasted_iota(jnp.int32, (BR, E), 1)
    ga0, gb0, oha0, ohb0 = _top2(lg_ref[0], lanes)
    ga1, gb1, oha1, ohb1 = _top2(lg_ref[1], lanes)

    gsum = oha0 + ohb0 + oha1 + ohb1
    r = lax.broadcasted_iota(jnp.int32, (BR, BR), 0)
    c = lax.broadcasted_iota(jnp.int32, (BR, BR), 1)
    lstrict = (c < r).astype(jnp.float32)
    f = lax.dot_general(lstrict, gsum, (((1,), (0,)), ((), ())),
                        preferred_element_type=jnp.float32) + acc_ref[...]
    corr = oha0 + ohb0

    def emit(b, oh1, oh2, g0, g1, fb):
        pos0 = jnp.sum(fb * oh1, axis=-1, keepdims=True).astype(jnp.int32)
        pos1 = jnp.sum(fb * oh2, axis=-1, keepdims=True).astype(jnp.int32)
        i1 = jnp.sum(
            lanes * oh1.astype(jnp.int32), axis=-1, keepdims=True)
        i2 = jnp.sum(
            lanes * oh2.astype(jnp.int32), axis=-1, keepdims=True)
        kp0 = (pos0 < C).astype(jnp.int32)
        kp1 = (pos1 < C).astype(jnp.int32)
        s0 = i1 * C + jnp.minimum(pos0, C - 1)
        s1 = i2 * C + jnp.minimum(pos1, C - 1)
        g0_ref[b] = g0
        g1_ref[b] = g1
        kp0_ref[b] = kp0
        kp1_ref[b] = kp1
        src0_ref[b] = s0
        src1_ref[b] = s1
        dst0_ref[b] = jnp.where(kp0 > 0, s0, TRASH)
        dst1_ref[b] = jnp.where(kp1 > 0, s1, TRASH)

    emit(0, oha0, ohb0, ga0, gb0, f)
    emit(1, oha1, ohb1, ga1, gb1, f + corr)

    acc_ref[...] = acc_ref[...] + jnp.sum(gsum, axis=0, keepdims=True)


def _router(logits2):
    BR = 512
    grid = (S // BR,)
    spec_d = pl.BlockSpec((B, BR, 1), lambda i: (0, i, 0))
    f32 = jax.ShapeDtypeStruct((B, S, 1), jnp.float32)
    i32 = jax.ShapeDtypeStruct((B, S, 1), jnp.int32)
    return pl.pallas_call(
        _router_body,
        grid=grid,
        in_specs=[pl.BlockSpec((B, BR, E), lambda i: (0, i, 0))],
        out_specs=[spec_d] * 8,
        out_shape=[f32, f32, i32, i32, i32, i32, i32, i32],
        scratch_shapes=[pltpu.VMEM((1, E), jnp.float32)],
    )(logits2)


def _sc_dispatch(x1, dst_idx):
    mesh = plsc.VectorSubcoreMesh(core_axis_name="c", subcore_axis_name="s")

    @functools.partial(
        pl.kernel,
        out_type=jax.ShapeDtypeStruct((BUF_ROWS, D), jnp.float32),
        mesh=mesh,
        scratch_types=[
            pltpu.VMEM((2 * NCH, CH), jnp.int32),
            pltpu.VMEM((3, CH, D), jnp.float32),
            pltpu.SemaphoreType.DMA,
            pltpu.SemaphoreType.DMA,
        ],
    )
    def k(x1_hbm, dst_hbm, buf_hbm, idx_v, rows_v, lsem, ssem):
        nc = 2
        wid = lax.axis_index("s") * nc + lax.axis_index("c")
        base = wid * TPW
        pltpu.sync_copy(dst_hbm.at[wid], idx_v)
        loads = {}
        scats = {}
        for c in range(min(3, NCH)):
            loads[c] = pltpu.async_copy(
                x1_hbm.at[pl.ds(base + c * CH, CH)], rows_v.at[c % 3], lsem)
        for c in range(NCH):
            b = c % 3
            if c >= 3:
                for h in scats[c - 3]:
                    h.wait()
                loads[c] = pltpu.async_copy(
                    x1_hbm.at[pl.ds(base + c * CH, CH)], rows_v.at[b], lsem)
            loads[c].wait()
            scats[c] = (
                pltpu.async_copy(rows_v.at[b], buf_hbm.at[idx_v.at[c]], ssem),
                pltpu.async_copy(rows_v.at[b], buf_hbm.at[idx_v.at[NCH + c]],
                                 ssem),
            )
        for c in range(max(0, NCH - 3), NCH):
            for h in scats[c]:
                h.wait()

    return k(x1, dst_idx)


def _sc_gather(ob, src_idx):
    mesh = plsc.VectorSubcoreMesh(core_axis_name="c", subcore_axis_name="s")

    @functools.partial(
        pl.kernel,
        out_type=jax.ShapeDtypeStruct((2 * T, D), jnp.float32),
        mesh=mesh,
        scratch_types=[
            pltpu.VMEM((2 * NCH, CH), jnp.int32),
            pltpu.VMEM((3, CH, D), jnp.float32),
            pltpu.SemaphoreType.DMA,
            pltpu.SemaphoreType.DMA,
        ],
    )
    def k(ob_hbm, src_hbm, comb_hbm, idx_v, rows_v, gsem, wsem):
        nc = 2
        wid = lax.axis_index("s") * nc + lax.axis_index("c")
        base = wid * TPW
        pltpu.sync_copy(src_hbm.at[wid], idx_v)
        ntot = 2 * NCH
        gaths = {}
        writes = {}

        def off(c):
            return (c // NCH) * T + base + (c % NCH) * CH

        for c in range(min(3, ntot)):
            gaths[c] = pltpu.async_copy(ob_hbm.at[idx_v.at[c]],
                                        rows_v.at[c % 3], gsem)
        for c in range(ntot):
            b = c % 3
            if c >= 3:
                writes[c - 3].wait()
                gaths[c] = pltpu.async_copy(ob_hbm.at[idx_v.at[c]],
                                            rows_v.at[b], gsem)
            gaths[c].wait()
            writes[c] = pltpu.async_copy(
                rows_v.at[b], comb_hbm.at[pl.ds(off(c), CH)], wsem)
        for c in range(max(0, ntot - 3), ntot):
            writes[c].wait()

    return k(ob, src_idx)


def _ffn_body(x_ref, w1_ref, b1_ref, w2_ref, b2_ref, o_ref):
    fb = pl.program_id(1)
    h = lax.dot_general(x_ref[...], w1_ref[0], (((1,), (0,)), ((), ())),
                        preferred_element_type=jnp.float32)
    h = jnp.maximum(h + b1_ref[0], 0.0)
    part = lax.dot_general(h, w2_ref[0], (((1,), (0,)), ((), ())),
                           preferred_element_type=jnp.float32)

    @pl.when(fb == 0)
    def _():
        o_ref[...] = part + b2_ref[0]

    @pl.when(fb > 0)
    def _():
        o_ref[...] = o_ref[...] + part


def _ffn(buf, W1, b1, W2, b2):
    BF = 1024
    nfb = DFF // BF
    grid = (E, nfb)
    return pl.pallas_call(
        _ffn_body,
        grid=grid,
        in_specs=[
            pl.BlockSpec((C, D), lambda e, fb: (e, 0)),
            pl.BlockSpec((1, D, BF), lambda e, fb: (e, 0, fb)),
            pl.BlockSpec((1, 1, BF), lambda e, fb: (e, 0, fb)),
            pl.BlockSpec((1, BF, D), lambda e, fb: (e, fb, 0)),
            pl.BlockSpec((1, 1, D), lambda e, fb: (e, 0, 0)),
        ],
        out_specs=pl.BlockSpec((C, D), lambda e, fb: (e, 0)),
        out_shape=jax.ShapeDtypeStruct((E * C, D), jnp.float32),
    )(buf, W1, b1.reshape(E, 1, DFF), W2, b2.reshape(E, 1, D))


def _combine_body(x1a_ref, x1b_ref, c0a_ref, c0b_ref, c1a_ref, c1b_ref,
                  g0_ref, g1_ref, k0_ref, k1_ref, g_ref, b_ref, o_ref):
    def side(b, x1_ref, c0_ref, c1_ref):
        x1 = x1_ref[...]
        m0 = jnp.where(k0_ref[b] > 0, g0_ref[b] * c0_ref[...], 0.0)
        m1 = jnp.where(k1_ref[b] > 0, g1_ref[b] * c1_ref[...], 0.0)
        y = x1 + m0 + m1
        mu = jnp.mean(y, axis=-1, keepdims=True)
        d = y - mu
        var = jnp.mean(d * d, axis=-1, keepdims=True)
        return d * lax.rsqrt(var + 1e-5) * g_ref[...] + b_ref[...]

    y0 = side(0, x1a_ref, c0a_ref, c1a_ref)
    y1 = side(1, x1b_ref, c0b_ref, c1b_ref)
    o_ref[...] = jnp.concatenate([y0[:, None, :], y1[:, None, :]], axis=1)


def _combine_ln2(x1, comb, g0, g1, kp0, kp1, ln2_g, ln2_b):
    BR = 128
    nb = S // BR
    grid = (nb,)
    spec_d = pl.BlockSpec((B, BR, 1), lambda i: (0, i, 0))
    spec_row = lambda blk: pl.BlockSpec((BR, D), lambda i, b=blk: (b + i, 0))
    return pl.pallas_call(
        _combine_body,
        grid=grid,
        in_specs=[
            spec_row(0), spec_row(nb),
            spec_row(0), spec_row(nb),
            spec_row(2 * nb), spec_row(3 * nb),
            spec_d, spec_d, spec_d, spec_d,
            pl.BlockSpec((1, D), lambda i: (0, 0)),
            pl.BlockSpec((1, D), lambda i: (0, 0)),
        ],
        out_specs=pl.BlockSpec((BR, B, D), lambda i: (i, 0, 0)),
        out_shape=jax.ShapeDtypeStruct((S, B, D), jnp.float32),
    )(x1, x1, comb, comb, comb, comb, g0, g1, kp0, kp1,
      ln2_g.reshape(1, D), ln2_b.reshape(1, D))


def kernel(src, in_proj_w, in_proj_b, out_w, out_b, ln1_g, ln1_b, ln2_g,
           ln2_b, Wg, W1, b1, W2, b2):
    qkv = _qkv_proj(src, in_proj_w, in_proj_b)
    attn = _attention(qkv)
    x1, logits = _proj_ln1(attn, src, out_w, out_b, ln1_g, ln1_b, Wg)
    g0, g1, dst0, dst1, src0, src1, kp0, kp1 = _router(
        logits.reshape(B, S, E))

    def chunked(a):
        return a.reshape(NW, NCH, CH)

    dst_idx = jnp.concatenate([chunked(dst0), chunked(dst1)], axis=1)
    src_idx = jnp.concatenate([chunked(src0), chunked(src1)], axis=1)

    buf = _sc_dispatch(x1, dst_idx)
    ob = _ffn(buf, W1, b1, W2, b2)
    comb = _sc_gather(ob, src_idx)
    return _combine_ln2(x1, comb, g0, g1, kp0, kp1, ln2_g, ln2_b)
